# baseline (speedup 1.0000x reference)
import functools

import jax
import jax.numpy as jnp
from jax.experimental import pallas as pl
from jax.experimental.pallas import tpu as pltpu

NUM_EXPERTS = 8
TOP_K = 2
HIDDEN = 2048
INTER = 768
T = 2048

BLOCK_T = 256


def _routing_weights(logits, e):
    bt = logits.shape[0]
    idx = jax.lax.broadcasted_iota(jnp.int32, (1, NUM_EXPERTS), 1)
    lj = logits[:, None, :]
    lk = logits[:, :, None]
    gt = (lk > lj).astype(jnp.int32)
    eq = ((lk == lj) & (idx[:, :, None] < idx)).astype(jnp.int32)
    rank = jnp.sum(gt + eq, axis=1)
    top2 = rank < TOP_K
    m1 = jnp.max(logits, axis=1, keepdims=True)
    ex = jnp.exp(logits - m1)
    denom = jnp.sum(jnp.where(top2, ex, 0.0), axis=1, keepdims=True)
    w_all = jnp.where(top2, ex / denom, 0.0)
    onehot = (idx == e).astype(w_all.dtype)
    return jnp.sum(w_all * onehot, axis=1)


def _dense_body(logits_ref, x_ref, gw_ref, uw_ref, dw_ref, acc_ref, out_ref):
    e = pl.program_id(1)
    x = x_ref[...]
    g = jnp.dot(x, gw_ref[0], preferred_element_type=jnp.float32)
    u = jnp.dot(x, uw_ref[0], preferred_element_type=jnp.float32)
    h = g * jax.lax.logistic(g) * u
    y = jnp.dot(h, dw_ref[0], preferred_element_type=jnp.float32)
    w = _routing_weights(logits_ref[...], e)

    @pl.when(e == 0)
    def _():
        acc_ref[...] = jnp.zeros_like(acc_ref)

    acc_ref[...] += w[:, None] * y

    @pl.when(e == NUM_EXPERTS - 1)
    def _():
        out_ref[...] = acc_ref[...]


@jax.jit
def kernel(hidden_states, router_logits, gate_w, up_w, down_w):
    nt = T // BLOCK_T
    grid = (nt, NUM_EXPERTS)
    out = pl.pallas_call(
        _dense_body,
        grid=grid,
        in_specs=[
            pl.BlockSpec((BLOCK_T, NUM_EXPERTS), lambda t, e: (t, 0)),
            pl.BlockSpec((BLOCK_T, HIDDEN), lambda t, e: (t, 0)),
            pl.BlockSpec((1, HIDDEN, INTER), lambda t, e: (e, 0, 0)),
            pl.BlockSpec((1, HIDDEN, INTER), lambda t, e: (e, 0, 0)),
            pl.BlockSpec((1, INTER, HIDDEN), lambda t, e: (e, 0, 0)),
        ],
        out_specs=pl.BlockSpec((BLOCK_T, HIDDEN), lambda t, e: (t, 0)),
        out_shape=jax.ShapeDtypeStruct((T, HIDDEN), jnp.float32),
        scratch_shapes=[pltpu.VMEM((BLOCK_T, HIDDEN), jnp.float32)],
    )(router_logits, hidden_states, gate_w, up_w, down_w)
    return out


# speedup vs baseline: 1.1417x; 1.1417x over previous
import functools

import jax
import jax.numpy as jnp
from jax import lax
from jax.experimental import pallas as pl
from jax.experimental.pallas import tpu as pltpu
from jax.experimental.pallas import tpu_sc as plsc

NUM_EXPERTS = 8
TOP_K = 2
HIDDEN = 2048
INTER = 768
T = 2048

BM = 128
P = T * TOP_K + NUM_EXPERTS * BM
NB = P // BM
NBPAD = 64

NW = 32
ROWS_PER_W = P // NW
TOK_PER_W = T // NW


def _routing_body(logits_ref, slotk_ref, w2_ref, be_ref):
    l = logits_ref[...]
    ik = lax.broadcasted_iota(jnp.int32, (1, NUM_EXPERTS, NUM_EXPERTS), 1)
    ij = lax.broadcasted_iota(jnp.int32, (1, NUM_EXPERTS, NUM_EXPERTS), 2)
    lk = l[:, :, None]
    lj = l[:, None, :]
    gt = (lk > lj).astype(jnp.int32)
    eq = ((lk == lj) & (ik < ij)).astype(jnp.int32)
    rank = jnp.sum(gt + eq, axis=1)
    top2 = rank < TOP_K
    m1 = jnp.max(l, axis=1, keepdims=True)
    ex = jnp.exp(l - m1)
    denom = jnp.sum(jnp.where(top2, ex, 0.0), axis=1, keepdims=True)
    w_all = jnp.where(top2, ex / denom, 0.0)

    ti = lax.broadcasted_iota(jnp.int32, (T, T), 0)
    tj = lax.broadcasted_iota(jnp.int32, (T, T), 1)
    lower = (ti >= tj).astype(jnp.float32)
    maskf = top2.astype(jnp.float32)
    csum = jnp.dot(lower, maskf, preferred_element_type=jnp.float32)
    r_in_e = csum.astype(jnp.int32) - 1

    counts = csum[T - 1:T, :].astype(jnp.int32)
    padded = ((counts + (BM - 1)) // BM) * BM
    ei = lax.broadcasted_iota(jnp.int32, (NUM_EXPERTS, NUM_EXPERTS), 0)
    ej = lax.broadcasted_iota(jnp.int32, (NUM_EXPERTS, NUM_EXPERTS), 1)
    strict = (ei < ej).astype(jnp.float32)
    base = jnp.dot(padded.astype(jnp.float32), strict,
                   preferred_element_type=jnp.float32).astype(jnp.int32)

    slot = base + r_in_e

    karr = lax.broadcasted_iota(jnp.int32, (TOP_K, 1, 1), 0)
    onek = rank[None, :, :] == karr
    slotk = jnp.sum(jnp.where(onek, slot[None, :, :], 0), axis=2)
    w2 = jnp.sum(jnp.where(onek, w_all[None, :, :], 0.0), axis=2)
    slotk_ref[...] = slotk
    w2_ref[...] = w2

    endb = (base + padded) // BM
    bvals = lax.broadcasted_iota(jnp.int32, (1, NBPAD), 1)
    nfull = jnp.sum((bvals[:, :, None] >= endb[:, None, :]).astype(jnp.int32),
                    axis=2)
    be_ref[...] = jnp.minimum(nfull, NUM_EXPERTS - 1)


def _routing_call(router_logits):
    return pl.pallas_call(
        _routing_body,
        out_shape=[
            jax.ShapeDtypeStruct((TOP_K, T), jnp.int32),
            jax.ShapeDtypeStruct((TOP_K, T), jnp.float32),
            jax.ShapeDtypeStruct((1, NBPAD), jnp.int32),
        ],
    )(router_logits)


def _dispatch_body(x_hbm, slotk_hbm, w2_hbm, w_out, xs_out,
                   slotk_v, w2_v, ids_v, wv_v, buf, sem):
    wid = lax.axis_index("s") * 2 + lax.axis_index("c")
    pltpu.sync_copy(slotk_hbm, slotk_v)
    pltpu.sync_copy(w2_hbm, w2_v)

    zi = jnp.zeros((16,), jnp.int32)
    zf = jnp.zeros((16,), jnp.float32)

    def initbody(i, carry):
        ids_v[pl.ds(i * 16, 16)] = zi
        wv_v[pl.ds(i * 16, 16)] = zf
        return carry

    lax.fori_loop(0, P // 16, initbody, 0)

    iota16 = lax.broadcasted_iota(jnp.int32, (16,), 0)

    def make_scat(k):
        def body(c, carry):
            sl = slotk_v[k, pl.ds(c * 16, 16)]
            tok = c * 16 + iota16
            wv = w2_v[k, pl.ds(c * 16, 16)]
            plsc.store_scatter(ids_v, [sl], tok)
            plsc.store_scatter(wv_v, [sl], wv)
            return carry
        return body

    lax.fori_loop(0, T // 16, make_scat(0), 0)
    lax.fori_loop(0, T // 16, make_scat(1), 0)

    @pl.when(wid == 0)
    def _():
        pltpu.sync_copy(wv_v, w_out)

    base = wid * ROWS_PER_W

    def gbody(c, carry):
        idxv = ids_v[pl.ds(base + c * 16, 16)]
        pltpu.async_copy(x_hbm.at[idxv], buf, sem).wait()
        pltpu.sync_copy(buf, xs_out.at[pl.ds(base + c * 16, 16)])
        return carry

    lax.fori_loop(0, ROWS_PER_W // 16, gbody, 0)


def _dispatch_call(hidden_states, slotk, w2):
    mesh = plsc.VectorSubcoreMesh(core_axis_name="c", subcore_axis_name="s")
    f = functools.partial(
        pl.kernel, _dispatch_body, mesh=mesh,
        out_type=[
            jax.ShapeDtypeStruct((P,), jnp.float32),
            jax.ShapeDtypeStruct((P, HIDDEN), jnp.float32),
        ],
        scratch_types=[
            pltpu.VMEM((TOP_K, T), jnp.int32),
            pltpu.VMEM((TOP_K, T), jnp.float32),
            pltpu.VMEM((P,), jnp.int32),
            pltpu.VMEM((P,), jnp.float32),
            pltpu.VMEM((16, HIDDEN), jnp.float32),
            pltpu.SemaphoreType.DMA,
        ],
        compiler_params=pltpu.CompilerParams(needs_layout_passes=False),
    )()
    return f(hidden_states, slotk, w2)


def _expert_body(be_ref, x_ref, w_ref, gw_ref, uw_ref, dw_ref, y_ref):
    x = x_ref[...]
    g = jnp.dot(x, gw_ref[0], preferred_element_type=jnp.float32)
    u = jnp.dot(x, uw_ref[0], preferred_element_type=jnp.float32)
    h = g * lax.logistic(g) * u
    h = h * w_ref[0, 0][:, None]
    y_ref[...] = jnp.dot(h, dw_ref[0], preferred_element_type=jnp.float32)


def _expert_call(be, x_sorted, w2d, gate_w, up_w, down_w):
    grid_spec = pltpu.PrefetchScalarGridSpec(
        num_scalar_prefetch=1,
        grid=(NB,),
        in_specs=[
            pl.BlockSpec((BM, HIDDEN), lambda b, be: (b, 0)),
            pl.BlockSpec((1, 1, BM), lambda b, be: (b, 0, 0)),
            pl.BlockSpec((1, HIDDEN, INTER), lambda b, be: (be[b], 0, 0)),
            pl.BlockSpec((1, HIDDEN, INTER), lambda b, be: (be[b], 0, 0)),
            pl.BlockSpec((1, INTER, HIDDEN), lambda b, be: (be[b], 0, 0)),
        ],
        out_specs=pl.BlockSpec((BM, HIDDEN), lambda b, be: (b, 0)),
    )
    return pl.pallas_call(
        _expert_body,
        grid_spec=grid_spec,
        out_shape=jax.ShapeDtypeStruct((P, HIDDEN), jnp.float32),
    )(be, x_sorted, w2d, gate_w, up_w, down_w)


def _combine_body(y_hbm, slotk_hbm, out_hbm, idx_v, a_v, o_v, s1, s2):
    wid = lax.axis_index("s") * 2 + lax.axis_index("c")
    t0 = wid * TOK_PER_W
    pltpu.sync_copy(slotk_hbm, idx_v)

    def body(c, carry):
        i0 = idx_v.at[0, pl.ds(t0 + c * 16, 16)]
        i1 = idx_v.at[1, pl.ds(t0 + c * 16, 16)]
        cp0 = pltpu.async_copy(y_hbm.at[i0], o_v, s1)
        cp1 = pltpu.async_copy(y_hbm.at[i1], a_v, s2)
        cp0.wait()
        cp1.wait()
        for r in range(16):
            def colbody(q, carry2, r=r):
                s = pl.ds(q * 16, 16)
                o_v[r, s] = o_v[r, s] + a_v[r, s]
                return carry2
            lax.fori_loop(0, HIDDEN // 16, colbody, 0)
        pltpu.sync_copy(o_v, out_hbm.at[pl.ds(t0 + c * 16, 16)])
        return carry

    lax.fori_loop(0, TOK_PER_W // 16, body, 0)


def _combine_call(y_sorted, slotk):
    mesh = plsc.VectorSubcoreMesh(core_axis_name="c", subcore_axis_name="s")
    f = functools.partial(
        pl.kernel, _combine_body, mesh=mesh,
        out_type=jax.ShapeDtypeStruct((T, HIDDEN), jnp.float32),
        scratch_types=[
            pltpu.VMEM((TOP_K, T), jnp.int32),
            pltpu.VMEM((16, HIDDEN), jnp.float32),
            pltpu.VMEM((16, HIDDEN), jnp.float32),
            pltpu.SemaphoreType.DMA,
            pltpu.SemaphoreType.DMA,
        ],
    )()
    return f(y_sorted, slotk)


@jax.jit
def kernel(hidden_states, router_logits, gate_w, up_w, down_w):
    slotk, w2, be2d = _routing_call(router_logits)
    be = be2d.reshape(-1)[:NB]
    w_sorted, x_sorted = _dispatch_call(hidden_states, slotk, w2)
    y_sorted = _expert_call(be, x_sorted, w_sorted.reshape(NB, 1, BM),
                            gate_w, up_w, down_w)
    return _combine_call(y_sorted, slotk)


# speedup vs baseline: 1.5863x; 1.3895x over previous
import functools

import jax
import jax.numpy as jnp
from jax import lax
from jax.experimental import pallas as pl
from jax.experimental.pallas import tpu as pltpu
from jax.experimental.pallas import tpu_sc as plsc

NUM_EXPERTS = 8
TOP_K = 2
HIDDEN = 2048
INTER = 768
T = 2048

BM = 128
P = T * TOP_K + NUM_EXPERTS * BM
NB = P // BM
NBPAD = 64

NW = 32
ROWS_PER_W = P // NW
TOK_PER_W = T // NW


def _routing_body(logits_ref, slotk_ref, w2_ref, be_ref):
    l = logits_ref[...]
    ik = lax.broadcasted_iota(jnp.int32, (1, NUM_EXPERTS, NUM_EXPERTS), 1)
    ij = lax.broadcasted_iota(jnp.int32, (1, NUM_EXPERTS, NUM_EXPERTS), 2)
    lk = l[:, :, None]
    lj = l[:, None, :]
    gt = (lk > lj).astype(jnp.int32)
    eq = ((lk == lj) & (ik < ij)).astype(jnp.int32)
    rank = jnp.sum(gt + eq, axis=1)
    top2 = rank < TOP_K
    m1 = jnp.max(l, axis=1, keepdims=True)
    ex = jnp.exp(l - m1)
    denom = jnp.sum(jnp.where(top2, ex, 0.0), axis=1, keepdims=True)
    w_all = jnp.where(top2, ex / denom, 0.0)

    ti = lax.broadcasted_iota(jnp.int32, (T, T), 0)
    tj = lax.broadcasted_iota(jnp.int32, (T, T), 1)
    lower = (ti >= tj).astype(jnp.float32)
    maskf = top2.astype(jnp.float32)
    csum = jnp.dot(lower, maskf, preferred_element_type=jnp.float32)
    r_in_e = csum.astype(jnp.int32) - 1

    counts = csum[T - 1:T, :].astype(jnp.int32)
    padded = ((counts + (BM - 1)) // BM) * BM
    ei = lax.broadcasted_iota(jnp.int32, (NUM_EXPERTS, NUM_EXPERTS), 0)
    ej = lax.broadcasted_iota(jnp.int32, (NUM_EXPERTS, NUM_EXPERTS), 1)
    strict = (ei < ej).astype(jnp.float32)
    base = jnp.dot(padded.astype(jnp.float32), strict,
                   preferred_element_type=jnp.float32).astype(jnp.int32)

    slot = base + r_in_e

    karr = lax.broadcasted_iota(jnp.int32, (TOP_K, 1, 1), 0)
    onek = rank[None, :, :] == karr
    slotk = jnp.sum(jnp.where(onek, slot[None, :, :], 0), axis=2)
    w2 = jnp.sum(jnp.where(onek, w_all[None, :, :], 0.0), axis=2)
    slotk_ref[...] = slotk
    w2_ref[...] = w2

    endb = (base + padded) // BM
    bvals = lax.broadcasted_iota(jnp.int32, (1, NBPAD), 1)
    nfull = jnp.sum((bvals[:, :, None] >= endb[:, None, :]).astype(jnp.int32),
                    axis=2)
    be_ref[...] = jnp.minimum(nfull, NUM_EXPERTS - 1)


def _routing_call(router_logits):
    return pl.pallas_call(
        _routing_body,
        out_shape=[
            jax.ShapeDtypeStruct((TOP_K, T), jnp.int32),
            jax.ShapeDtypeStruct((TOP_K, T), jnp.float32),
            jax.ShapeDtypeStruct((1, NBPAD), jnp.int32),
        ],
    )(router_logits)


DCH = 32
NDCH = TOK_PER_W // DCH


def _dispatch_body(x_hbm, slotk_hbm, w2_hbm, w_out, xs_out,
                   idx3_v, rows_v, slotk_v, w2_v, wv_v, s0, s1):
    wid = lax.axis_index("s") * 2 + lax.axis_index("c")
    t0 = wid * TOK_PER_W

    for k in range(TOP_K):
        for c in range(NDCH):
            pltpu.sync_copy(slotk_hbm.at[k, pl.ds(t0 + c * DCH, DCH)],
                            idx3_v.at[k, c])

    @pl.when(wid == 0)
    def _():
        pltpu.sync_copy(slotk_hbm, slotk_v)
        pltpu.sync_copy(w2_hbm, w2_v)
        zf = jnp.zeros((16,), jnp.float32)

        def initbody(i, carry):
            wv_v[pl.ds(i * 16, 16)] = zf
            return carry

        lax.fori_loop(0, P // 16, initbody, 0)

        def make_scat(k):
            def body(c, carry):
                sl = slotk_v[k, pl.ds(c * 16, 16)]
                wv = w2_v[k, pl.ds(c * 16, 16)]
                plsc.store_scatter(wv_v, [sl], wv)
                return carry
            return body

        lax.fori_loop(0, T // 16, make_scat(0), 0)
        lax.fori_loop(0, T // 16, make_scat(1), 0)
        pltpu.sync_copy(wv_v, w_out)

    for c in range(NDCH):
        pltpu.sync_copy(x_hbm.at[pl.ds(t0 + c * DCH, DCH)], rows_v)
        cp0 = pltpu.async_copy(rows_v, xs_out.at[idx3_v.at[0, c]], s0)
        cp1 = pltpu.async_copy(rows_v, xs_out.at[idx3_v.at[1, c]], s1)
        cp0.wait()
        cp1.wait()


def _dispatch_call(hidden_states, slotk, w2):
    mesh = plsc.VectorSubcoreMesh(core_axis_name="c", subcore_axis_name="s")
    f = functools.partial(
        pl.kernel, _dispatch_body, mesh=mesh,
        out_type=[
            jax.ShapeDtypeStruct((P,), jnp.float32),
            jax.ShapeDtypeStruct((P, HIDDEN), jnp.float32),
        ],
        scratch_types=[
            pltpu.VMEM((TOP_K, NDCH, DCH), jnp.int32),
            pltpu.VMEM((DCH, HIDDEN), jnp.float32),
            pltpu.VMEM((TOP_K, T), jnp.int32),
            pltpu.VMEM((TOP_K, T), jnp.float32),
            pltpu.VMEM((P,), jnp.float32),
            pltpu.SemaphoreType.DMA,
            pltpu.SemaphoreType.DMA,
        ],
        compiler_params=pltpu.CompilerParams(needs_layout_passes=False),
    )()
    return f(hidden_states, slotk, w2)


def _expert_body(be_ref, x_ref, w_ref, gw_ref, uw_ref, dw_ref, y_ref):
    x = x_ref[...]
    g = jnp.dot(x, gw_ref[0], preferred_element_type=jnp.float32)
    u = jnp.dot(x, uw_ref[0], preferred_element_type=jnp.float32)
    h = g * lax.logistic(g) * u
    h = h * w_ref[0, 0][:, None]
    y_ref[...] = jnp.dot(h, dw_ref[0], preferred_element_type=jnp.float32)


def _expert_call(be, x_sorted, w2d, gate_w, up_w, down_w):
    grid_spec = pltpu.PrefetchScalarGridSpec(
        num_scalar_prefetch=1,
        grid=(NB,),
        in_specs=[
            pl.BlockSpec((BM, HIDDEN), lambda b, be: (b, 0)),
            pl.BlockSpec((1, 1, BM), lambda b, be: (b, 0, 0)),
            pl.BlockSpec((1, HIDDEN, INTER), lambda b, be: (be[b], 0, 0)),
            pl.BlockSpec((1, HIDDEN, INTER), lambda b, be: (be[b], 0, 0)),
            pl.BlockSpec((1, INTER, HIDDEN), lambda b, be: (be[b], 0, 0)),
        ],
        out_specs=pl.BlockSpec((BM, HIDDEN), lambda b, be: (b, 0)),
    )
    return pl.pallas_call(
        _expert_body,
        grid_spec=grid_spec,
        out_shape=jax.ShapeDtypeStruct((P, HIDDEN), jnp.float32),
    )(be, x_sorted, w2d, gate_w, up_w, down_w)


def _combine_body(y_hbm, slotk_hbm, out_hbm, idx_v, a_v, o_v, s1, s2):
    wid = lax.axis_index("s") * 2 + lax.axis_index("c")
    t0 = wid * TOK_PER_W
    pltpu.sync_copy(slotk_hbm, idx_v)

    def body(c, carry):
        i0 = idx_v.at[0, pl.ds(t0 + c * 16, 16)]
        i1 = idx_v.at[1, pl.ds(t0 + c * 16, 16)]
        cp0 = pltpu.async_copy(y_hbm.at[i0], o_v, s1)
        cp1 = pltpu.async_copy(y_hbm.at[i1], a_v, s2)
        cp0.wait()
        cp1.wait()
        for r in range(16):
            def colbody(q, carry2, r=r):
                s = pl.ds(q * 16, 16)
                o_v[r, s] = o_v[r, s] + a_v[r, s]
                return carry2
            lax.fori_loop(0, HIDDEN // 16, colbody, 0)
        pltpu.sync_copy(o_v, out_hbm.at[pl.ds(t0 + c * 16, 16)])
        return carry

    lax.fori_loop(0, TOK_PER_W // 16, body, 0)


def _combine_call(y_sorted, slotk):
    mesh = plsc.VectorSubcoreMesh(core_axis_name="c", subcore_axis_name="s")
    f = functools.partial(
        pl.kernel, _combine_body, mesh=mesh,
        out_type=jax.ShapeDtypeStruct((T, HIDDEN), jnp.float32),
        scratch_types=[
            pltpu.VMEM((TOP_K, T), jnp.int32),
            pltpu.VMEM((16, HIDDEN), jnp.float32),
            pltpu.VMEM((16, HIDDEN), jnp.float32),
            pltpu.SemaphoreType.DMA,
            pltpu.SemaphoreType.DMA,
        ],
    )()
    return f(y_sorted, slotk)


@jax.jit
def kernel(hidden_states, router_logits, gate_w, up_w, down_w):
    slotk, w2, be2d = _routing_call(router_logits)
    be = be2d.reshape(-1)[:NB]
    w_sorted, x_sorted = _dispatch_call(hidden_states, slotk, w2)
    y_sorted = _expert_call(be, x_sorted, w_sorted.reshape(NB, 1, BM),
                            gate_w, up_w, down_w)
    return _combine_call(y_sorted, slotk)


# speedup vs baseline: 1.7443x; 1.0996x over previous
import functools

import jax
import jax.numpy as jnp
from jax import lax
from jax.experimental import pallas as pl
from jax.experimental.pallas import tpu as pltpu
from jax.experimental.pallas import tpu_sc as plsc

NUM_EXPERTS = 8
TOP_K = 2
HIDDEN = 2048
INTER = 768
T = 2048

BM = 128
P = T * TOP_K + NUM_EXPERTS * BM
NB = P // BM
NBPAD = 64

NW = 32
ROWS_PER_W = P // NW
TOK_PER_W = T // NW


def _routing_body(logits_ref, slotk_ref, w2_ref, be_ref):
    l = logits_ref[...]
    ik = lax.broadcasted_iota(jnp.int32, (1, NUM_EXPERTS, NUM_EXPERTS), 1)
    ij = lax.broadcasted_iota(jnp.int32, (1, NUM_EXPERTS, NUM_EXPERTS), 2)
    lk = l[:, :, None]
    lj = l[:, None, :]
    gt = (lk > lj).astype(jnp.int32)
    eq = ((lk == lj) & (ik < ij)).astype(jnp.int32)
    rank = jnp.sum(gt + eq, axis=1)
    top2 = rank < TOP_K
    m1 = jnp.max(l, axis=1, keepdims=True)
    ex = jnp.exp(l - m1)
    denom = jnp.sum(jnp.where(top2, ex, 0.0), axis=1, keepdims=True)
    w_all = jnp.where(top2, ex / denom, 0.0)

    ti = lax.broadcasted_iota(jnp.int32, (T, T), 0)
    tj = lax.broadcasted_iota(jnp.int32, (T, T), 1)
    lower = (ti >= tj).astype(jnp.float32)
    maskf = top2.astype(jnp.float32)
    csum = jnp.dot(lower, maskf, preferred_element_type=jnp.float32)
    r_in_e = csum.astype(jnp.int32) - 1

    counts = csum[T - 1:T, :].astype(jnp.int32)
    padded = ((counts + (BM - 1)) // BM) * BM
    ei = lax.broadcasted_iota(jnp.int32, (NUM_EXPERTS, NUM_EXPERTS), 0)
    ej = lax.broadcasted_iota(jnp.int32, (NUM_EXPERTS, NUM_EXPERTS), 1)
    strict = (ei < ej).astype(jnp.float32)
    base = jnp.dot(padded.astype(jnp.float32), strict,
                   preferred_element_type=jnp.float32).astype(jnp.int32)

    slot = base + r_in_e

    karr = lax.broadcasted_iota(jnp.int32, (TOP_K, 1, 1), 0)
    onek = rank[None, :, :] == karr
    slotk = jnp.sum(jnp.where(onek, slot[None, :, :], 0), axis=2)
    w2 = jnp.sum(jnp.where(onek, w_all[None, :, :], 0.0), axis=2)
    slotk_ref[...] = slotk
    w2_ref[...] = w2

    endb = (base + padded) // BM
    bvals = lax.broadcasted_iota(jnp.int32, (1, NBPAD), 1)
    nfull = jnp.sum((bvals[:, :, None] >= endb[:, None, :]).astype(jnp.int32),
                    axis=2)
    be_ref[...] = jnp.minimum(nfull, NUM_EXPERTS - 1)


def _routing_call(router_logits):
    return pl.pallas_call(
        _routing_body,
        out_shape=[
            jax.ShapeDtypeStruct((TOP_K, T), jnp.int32),
            jax.ShapeDtypeStruct((TOP_K, T), jnp.float32),
            jax.ShapeDtypeStruct((1, NBPAD), jnp.int32),
        ],
    )(router_logits)


DCH = 32
NDCH = TOK_PER_W // DCH


def _dispatch_body(x_hbm, slotk_hbm, w2_hbm, w_out, xs_out,
                   idx3_v, rows_v, slotk_v, w2_v, wv_v, s0, s1):
    wid = lax.axis_index("s") * 2 + lax.axis_index("c")
    t0 = wid * TOK_PER_W

    for k in range(TOP_K):
        for c in range(NDCH):
            pltpu.sync_copy(slotk_hbm.at[k, pl.ds(t0 + c * DCH, DCH)],
                            idx3_v.at[k, c])

    @pl.when(wid == 0)
    def _():
        pltpu.sync_copy(slotk_hbm, slotk_v)
        pltpu.sync_copy(w2_hbm, w2_v)
        zf = jnp.zeros((16,), jnp.float32)

        def initbody(i, carry):
            wv_v[pl.ds(i * 16, 16)] = zf
            return carry

        lax.fori_loop(0, P // 16, initbody, 0)

        def make_scat(k):
            def body(c, carry):
                sl = slotk_v[k, pl.ds(c * 16, 16)]
                wv = w2_v[k, pl.ds(c * 16, 16)]
                plsc.store_scatter(wv_v, [sl], wv)
                return carry
            return body

        lax.fori_loop(0, T // 16, make_scat(0), 0)
        lax.fori_loop(0, T // 16, make_scat(1), 0)
        pltpu.sync_copy(wv_v, w_out)

    for c in range(NDCH):
        pltpu.sync_copy(x_hbm.at[pl.ds(t0 + c * DCH, DCH)], rows_v)
        cp0 = pltpu.async_copy(rows_v, xs_out.at[idx3_v.at[0, c]], s0)
        cp1 = pltpu.async_copy(rows_v, xs_out.at[idx3_v.at[1, c]], s1)
        cp0.wait()
        cp1.wait()


def _dispatch_call(hidden_states, slotk, w2):
    mesh = plsc.VectorSubcoreMesh(core_axis_name="c", subcore_axis_name="s")
    f = functools.partial(
        pl.kernel, _dispatch_body, mesh=mesh,
        out_type=[
            jax.ShapeDtypeStruct((P,), jnp.float32),
            jax.ShapeDtypeStruct((P, HIDDEN), jnp.float32),
        ],
        scratch_types=[
            pltpu.VMEM((TOP_K, NDCH, DCH), jnp.int32),
            pltpu.VMEM((DCH, HIDDEN), jnp.float32),
            pltpu.VMEM((TOP_K, T), jnp.int32),
            pltpu.VMEM((TOP_K, T), jnp.float32),
            pltpu.VMEM((P,), jnp.float32),
            pltpu.SemaphoreType.DMA,
            pltpu.SemaphoreType.DMA,
        ],
        compiler_params=pltpu.CompilerParams(needs_layout_passes=False),
    )()
    return f(hidden_states, slotk, w2)


def _expert_body(be_ref, x_ref, w_ref, gw_ref, uw_ref, dw_ref, y_ref):
    x = x_ref[...]
    g = jnp.dot(x, gw_ref[0], preferred_element_type=jnp.float32)
    u = jnp.dot(x, uw_ref[0], preferred_element_type=jnp.float32)
    h = g * lax.logistic(g) * u
    h = h * w_ref[0, 0][:, None]
    y_ref[...] = jnp.dot(h, dw_ref[0], preferred_element_type=jnp.float32)


def _expert_call(be, x_sorted, w2d, gate_w, up_w, down_w):
    grid_spec = pltpu.PrefetchScalarGridSpec(
        num_scalar_prefetch=1,
        grid=(NB,),
        in_specs=[
            pl.BlockSpec((BM, HIDDEN), lambda b, be: (b, 0)),
            pl.BlockSpec((1, 1, BM), lambda b, be: (b, 0, 0)),
            pl.BlockSpec((1, HIDDEN, INTER), lambda b, be: (be[b], 0, 0)),
            pl.BlockSpec((1, HIDDEN, INTER), lambda b, be: (be[b], 0, 0)),
            pl.BlockSpec((1, INTER, HIDDEN), lambda b, be: (be[b], 0, 0)),
        ],
        out_specs=pl.BlockSpec((BM, HIDDEN), lambda b, be: (b, 0)),
    )
    return pl.pallas_call(
        _expert_body,
        grid_spec=grid_spec,
        out_shape=jax.ShapeDtypeStruct((P, HIDDEN), jnp.float32),
    )(be, x_sorted, w2d, gate_w, up_w, down_w)


def _combine_body(y_hbm, slotk_hbm, out_hbm, idx_v, a_v, o_v, s1, s2):
    wid = lax.axis_index("s") * 2 + lax.axis_index("c")
    t0 = wid * TOK_PER_W
    pltpu.sync_copy(slotk_hbm, idx_v)

    def body(c, carry):
        i0 = idx_v.at[0, pl.ds(t0 + c * 16, 16)]
        i1 = idx_v.at[1, pl.ds(t0 + c * 16, 16)]
        cp0 = pltpu.async_copy(y_hbm.at[i0], o_v, s1)
        cp1 = pltpu.async_copy(y_hbm.at[i1], a_v, s2)
        cp0.wait()
        cp1.wait()

        def colbody(q, carry2):
            s = pl.ds(q * 16, 16)
            for r in range(16):
                o_v[r, s] = o_v[r, s] + a_v[r, s]
            return carry2

        lax.fori_loop(0, HIDDEN // 16, colbody, 0)
        pltpu.sync_copy(o_v, out_hbm.at[pl.ds(t0 + c * 16, 16)])
        return carry

    lax.fori_loop(0, TOK_PER_W // 16, body, 0)


def _combine_call(y_sorted, slotk):
    mesh = plsc.VectorSubcoreMesh(core_axis_name="c", subcore_axis_name="s")
    f = functools.partial(
        pl.kernel, _combine_body, mesh=mesh,
        out_type=jax.ShapeDtypeStruct((T, HIDDEN), jnp.float32),
        scratch_types=[
            pltpu.VMEM((TOP_K, T), jnp.int32),
            pltpu.VMEM((16, HIDDEN), jnp.float32),
            pltpu.VMEM((16, HIDDEN), jnp.float32),
            pltpu.SemaphoreType.DMA,
            pltpu.SemaphoreType.DMA,
        ],
    )()
    return f(y_sorted, slotk)


@jax.jit
def kernel(hidden_states, router_logits, gate_w, up_w, down_w):
    slotk, w2, be2d = _routing_call(router_logits)
    be = be2d.reshape(-1)[:NB]
    w_sorted, x_sorted = _dispatch_call(hidden_states, slotk, w2)
    y_sorted = _expert_call(be, x_sorted, w_sorted.reshape(NB, 1, BM),
                            gate_w, up_w, down_w)
    return _combine_call(y_sorted, slotk)


# speedup vs baseline: 1.8134x; 1.0396x over previous
import functools

import jax
import jax.numpy as jnp
from jax import lax
from jax.experimental import pallas as pl
from jax.experimental.pallas import tpu as pltpu
from jax.experimental.pallas import tpu_sc as plsc

NUM_EXPERTS = 8
TOP_K = 2
HIDDEN = 2048
INTER = 768
T = 2048

BM = 128
P = T * TOP_K + NUM_EXPERTS * BM
NB = P // BM
NBPAD = 64

NW = 32
ROWS_PER_W = P // NW
TOK_PER_W = T // NW


def _routing_body(logits_ref, slotk_ref, w2_ref, be_ref):
    l = logits_ref[...]
    ik = lax.broadcasted_iota(jnp.int32, (1, NUM_EXPERTS, NUM_EXPERTS), 1)
    ij = lax.broadcasted_iota(jnp.int32, (1, NUM_EXPERTS, NUM_EXPERTS), 2)
    lk = l[:, :, None]
    lj = l[:, None, :]
    gt = (lk > lj).astype(jnp.int32)
    eq = ((lk == lj) & (ik < ij)).astype(jnp.int32)
    rank = jnp.sum(gt + eq, axis=1)
    top2 = rank < TOP_K
    m1 = jnp.max(l, axis=1, keepdims=True)
    ex = jnp.exp(l - m1)
    denom = jnp.sum(jnp.where(top2, ex, 0.0), axis=1, keepdims=True)
    w_all = jnp.where(top2, ex / denom, 0.0)

    ti = lax.broadcasted_iota(jnp.int32, (T, T), 0)
    tj = lax.broadcasted_iota(jnp.int32, (T, T), 1)
    lower = (ti >= tj).astype(jnp.float32)
    maskf = top2.astype(jnp.float32)
    csum = jnp.dot(lower, maskf, preferred_element_type=jnp.float32)
    r_in_e = csum.astype(jnp.int32) - 1

    counts = csum[T - 1:T, :].astype(jnp.int32)
    padded = ((counts + (BM - 1)) // BM) * BM
    ei = lax.broadcasted_iota(jnp.int32, (NUM_EXPERTS, NUM_EXPERTS), 0)
    ej = lax.broadcasted_iota(jnp.int32, (NUM_EXPERTS, NUM_EXPERTS), 1)
    strict = (ei < ej).astype(jnp.float32)
    base = jnp.dot(padded.astype(jnp.float32), strict,
                   preferred_element_type=jnp.float32).astype(jnp.int32)

    slot = base + r_in_e

    karr = lax.broadcasted_iota(jnp.int32, (TOP_K, 1, 1), 0)
    onek = rank[None, :, :] == karr
    slotk = jnp.sum(jnp.where(onek, slot[None, :, :], 0), axis=2)
    w2 = jnp.sum(jnp.where(onek, w_all[None, :, :], 0.0), axis=2)
    slotk_ref[...] = slotk
    w2_ref[...] = w2

    endb = (base + padded) // BM
    bvals = lax.broadcasted_iota(jnp.int32, (1, NBPAD), 1)
    nfull = jnp.sum((bvals[:, :, None] >= endb[:, None, :]).astype(jnp.int32),
                    axis=2)
    be_ref[...] = jnp.minimum(nfull, NUM_EXPERTS - 1)


def _routing_call(router_logits):
    return pl.pallas_call(
        _routing_body,
        out_shape=[
            jax.ShapeDtypeStruct((TOP_K, T), jnp.int32),
            jax.ShapeDtypeStruct((TOP_K, T), jnp.float32),
            jax.ShapeDtypeStruct((1, NBPAD), jnp.int32),
        ],
    )(router_logits)


DCH = 32
NDCH = TOK_PER_W // DCH


def _dispatch_body(x_hbm, slotk_hbm, w2_hbm, w_out, xs_out,
                   idx3_v, rows_v, slotk_v, w2_v, wv_v, s0, s1):
    wid = lax.axis_index("s") * 2 + lax.axis_index("c")
    t0 = wid * TOK_PER_W

    for k in range(TOP_K):
        for c in range(NDCH):
            pltpu.sync_copy(slotk_hbm.at[k, pl.ds(t0 + c * DCH, DCH)],
                            idx3_v.at[k, c])

    @pl.when(wid == 0)
    def _():
        pltpu.sync_copy(slotk_hbm, slotk_v)
        pltpu.sync_copy(w2_hbm, w2_v)
        zf = jnp.zeros((16,), jnp.float32)

        def initbody(i, carry):
            wv_v[pl.ds(i * 16, 16)] = zf
            return carry

        lax.fori_loop(0, P // 16, initbody, 0)

        def make_scat(k):
            def body(c, carry):
                sl = slotk_v[k, pl.ds(c * 16, 16)]
                wv = w2_v[k, pl.ds(c * 16, 16)]
                plsc.store_scatter(wv_v, [sl], wv)
                return carry
            return body

        lax.fori_loop(0, T // 16, make_scat(0), 0)
        lax.fori_loop(0, T // 16, make_scat(1), 0)
        pltpu.sync_copy(wv_v, w_out)

    for c in range(NDCH):
        pltpu.sync_copy(x_hbm.at[pl.ds(t0 + c * DCH, DCH)], rows_v)
        cp0 = pltpu.async_copy(rows_v, xs_out.at[idx3_v.at[0, c]], s0)
        cp1 = pltpu.async_copy(rows_v, xs_out.at[idx3_v.at[1, c]], s1)
        cp0.wait()
        cp1.wait()


def _dispatch_call(hidden_states, slotk, w2):
    mesh = plsc.VectorSubcoreMesh(core_axis_name="c", subcore_axis_name="s")
    f = functools.partial(
        pl.kernel, _dispatch_body, mesh=mesh,
        out_type=[
            jax.ShapeDtypeStruct((P,), jnp.float32),
            jax.ShapeDtypeStruct((P, HIDDEN), jnp.float32),
        ],
        scratch_types=[
            pltpu.VMEM((TOP_K, NDCH, DCH), jnp.int32),
            pltpu.VMEM((DCH, HIDDEN), jnp.float32),
            pltpu.VMEM((TOP_K, T), jnp.int32),
            pltpu.VMEM((TOP_K, T), jnp.float32),
            pltpu.VMEM((P,), jnp.float32),
            pltpu.SemaphoreType.DMA,
            pltpu.SemaphoreType.DMA,
        ],
        compiler_params=pltpu.CompilerParams(needs_layout_passes=False),
    )()
    return f(hidden_states, slotk, w2)


def _expert_body(be_ref, x_ref, w_ref, gw_ref, uw_ref, dw_ref, y_ref):
    x = x_ref[...]
    g = jnp.dot(x, gw_ref[0], preferred_element_type=jnp.float32)
    u = jnp.dot(x, uw_ref[0], preferred_element_type=jnp.float32)
    h = g * lax.logistic(g) * u
    h = h * w_ref[0, 0][:, None]
    y_ref[...] = jnp.dot(h, dw_ref[0], preferred_element_type=jnp.float32)


def _expert_call(be, x_sorted, w2d, gate_w, up_w, down_w):
    grid_spec = pltpu.PrefetchScalarGridSpec(
        num_scalar_prefetch=1,
        grid=(NB,),
        in_specs=[
            pl.BlockSpec((BM, HIDDEN), lambda b, be: (b, 0)),
            pl.BlockSpec((1, 1, BM), lambda b, be: (b, 0, 0)),
            pl.BlockSpec((1, HIDDEN, INTER), lambda b, be: (be[0, b], 0, 0)),
            pl.BlockSpec((1, HIDDEN, INTER), lambda b, be: (be[0, b], 0, 0)),
            pl.BlockSpec((1, INTER, HIDDEN), lambda b, be: (be[0, b], 0, 0)),
        ],
        out_specs=pl.BlockSpec((BM, HIDDEN), lambda b, be: (b, 0)),
    )
    return pl.pallas_call(
        _expert_body,
        grid_spec=grid_spec,
        out_shape=jax.ShapeDtypeStruct((P, HIDDEN), jnp.float32),
    )(be, x_sorted, w2d, gate_w, up_w, down_w)


CCH = 8
NCCH = TOK_PER_W // CCH


def _combine_body(y_hbm, slotk_hbm, out_hbm, idx_v,
                  o0, a0, o1, a1, s1, s2, s3, s4):
    wid = lax.axis_index("s") * 2 + lax.axis_index("c")
    t0 = wid * TOK_PER_W
    for k in range(TOP_K):
        pltpu.sync_copy(slotk_hbm.at[k, pl.ds(t0, TOK_PER_W)], idx_v.at[k])

    bufs = [(o0, a0, s1, s2), (o1, a1, s3, s4)]

    def issue(c, bi):
        o, a, so, sa = bufs[bi]
        i0 = idx_v.at[0, pl.ds(c * CCH, CCH)]
        i1 = idx_v.at[1, pl.ds(c * CCH, CCH)]
        return (pltpu.async_copy(y_hbm.at[i0], o, so),
                pltpu.async_copy(y_hbm.at[i1], a, sa))

    cps = [None, None]
    cps[0] = issue(0, 0)
    for c in range(NCCH):
        bi = c % 2
        if c + 1 < NCCH:
            cps[1 - bi] = issue(c + 1, 1 - bi)
        cp0, cp1 = cps[bi]
        cp0.wait()
        cp1.wait()
        o, a, _, _ = bufs[bi]

        def colbody(q, carry2, o=o, a=a):
            s = pl.ds(q * 16, 16)
            for r in range(CCH):
                o[r, s] = o[r, s] + a[r, s]
            return carry2

        lax.fori_loop(0, HIDDEN // 16, colbody, 0)
        pltpu.sync_copy(o, out_hbm.at[pl.ds(t0 + c * CCH, CCH)])


def _combine_call(y_sorted, slotk):
    mesh = plsc.VectorSubcoreMesh(core_axis_name="c", subcore_axis_name="s")
    f = functools.partial(
        pl.kernel, _combine_body, mesh=mesh,
        out_type=jax.ShapeDtypeStruct((T, HIDDEN), jnp.float32),
        scratch_types=[
            pltpu.VMEM((TOP_K, TOK_PER_W), jnp.int32),
            pltpu.VMEM((CCH, HIDDEN), jnp.float32),
            pltpu.VMEM((CCH, HIDDEN), jnp.float32),
            pltpu.VMEM((CCH, HIDDEN), jnp.float32),
            pltpu.VMEM((CCH, HIDDEN), jnp.float32),
            pltpu.SemaphoreType.DMA,
            pltpu.SemaphoreType.DMA,
            pltpu.SemaphoreType.DMA,
            pltpu.SemaphoreType.DMA,
        ],
    )()
    return f(y_sorted, slotk)


@jax.jit
def kernel(hidden_states, router_logits, gate_w, up_w, down_w):
    slotk, w2, be2d = _routing_call(router_logits)
    w_sorted, x_sorted = _dispatch_call(hidden_states, slotk, w2)
    y_sorted = _expert_call(be2d, x_sorted, w_sorted.reshape(NB, 1, BM),
                            gate_w, up_w, down_w)
    return _combine_call(y_sorted, slotk)
